# baseline (speedup 1.0000x reference)
import functools

import numpy as np

import jax
import jax.numpy as jnp
from jax import lax
from jax.experimental import pallas as pl
from jax.experimental.pallas import tpu as pltpu
from jax.experimental.pallas import tpu_sc as plsc

N = 10000
R = 20
D = 64
E = 640000

NR = N * R
NR_PAD = 200704
NR_SHARD = NR_PAD // 16
N_PAD = 10112
N_SHARD = N_PAD // 16

CHUNK = 256
NSUB = CHUNK // 128
NCHUNK = 2560
E_PAD = NCHUNK * CHUNK
NW = 32
CPW = NCHUNK // NW

_p32 = np.empty((32,), np.int32)
_p32[0::2] = np.arange(16)
_p32[1::2] = 16 + np.arange(16)
_P64 = np.concatenate([_p32, 32 + _p32])

_mesh = plsc.VectorSubcoreMesh(core_axis_name="c", subcore_axis_name="s",
                               num_cores=2, num_subcores=16)


def _cnt_body(npair_hbm, out_hbm, idx_v, ones_v, zst_v, cnt_sh, sem):
    cid = lax.axis_index("c")
    sid = lax.axis_index("s")
    wid = sid * 2 + cid

    def zb(i, carry):
        zst_v[pl.ds(i * 16, 16)] = jnp.zeros((16,), jnp.float32)
        return carry
    lax.fori_loop(0, NR_SHARD // 16, zb, 0)
    for j in range(8):
        ones_v[pl.ds(16 * j, 16)] = jnp.ones((16,), jnp.float32)
    pltpu.sync_copy(zst_v, cnt_sh.at[pl.ds(sid * NR_SHARD, NR_SHARD)])
    plsc.subcore_barrier()

    def cb(i, carry):
        c = wid * CPW + i
        pltpu.sync_copy(npair_hbm.at[c], idx_v)
        for j in range(NSUB):
            pltpu.sync_copy(ones_v, cnt_sh.at[idx_v.at[j]], add=True)
        return carry
    lax.fori_loop(0, CPW, cb, 0)
    plsc.subcore_barrier()
    pltpu.sync_copy(cnt_sh.at[pl.ds(sid * NR_SHARD, NR_SHARD)],
                    out_hbm.at[cid, pl.ds(sid * NR_SHARD, NR_SHARD)])


_cnt_call = functools.partial(
    pl.kernel,
    out_type=jax.ShapeDtypeStruct((2, NR_PAD), jnp.float32),
    mesh=_mesh,
    scratch_types=[
        pltpu.VMEM((NSUB, 128), jnp.int32),
        pltpu.VMEM((128,), jnp.float32),
        pltpu.VMEM((NR_SHARD,), jnp.float32),
        pltpu.VMEM_SHARED((NR_PAD,), jnp.float32),
        pltpu.SemaphoreType.DMA,
    ],
)(_cnt_body)


def _agg_body(idxall_hbm, h_hbm, out_hbm,
              i0, i1, i2, i3, i4, i5, i6, i7,
              rb0, rb1, rb2, rb3, o0, o1, agg_sh,
              is0, is1, is2, is3, is4, is5, is6, is7,
              gs0, gs1, gs2, gs3, ss0, ss1):
    cid = lax.axis_index("c")
    sid = lax.axis_index("s")
    wid = sid * 2 + cid
    idxs = (i0, i1, i2, i3, i4, i5, i6, i7)
    rbf = (rb0, rb1, rb2, rb3)
    outs = (o0, o1)
    isem = (is0, is1, is2, is3, is4, is5, is6, is7)
    gsem = (gs0, gs1, gs2, gs3)
    ssem = (ss0, ss1)
    c0 = wid * CPW

    def zr(k, carry):
        for j in range(4):
            o0[k, pl.ds(16 * j, 16)] = jnp.zeros((16,), jnp.float32)
        return carry
    lax.fori_loop(0, CHUNK, zr, 0)
    base = sid * N_SHARD
    pltpu.sync_copy(o0, agg_sh.at[pl.ds(base, CHUNK)])
    pltpu.sync_copy(o0, agg_sh.at[pl.ds(base + CHUNK, CHUNK)])
    pltpu.sync_copy(o0.at[pl.ds(0, N_SHARD - 2 * CHUNK)],
                    agg_sh.at[pl.ds(base + 2 * CHUNK, N_SHARD - 2 * CHUNK)])
    plsc.subcore_barrier()

    def issue_idx(c, q):
        pltpu.async_copy(idxall_hbm.at[c], idxs[q], isem[q])

    def wait_idx(q):
        pltpu.make_async_copy(idxall_hbm.at[0], idxs[q], isem[q]).wait()

    def fire_gathers(b, q):
        for j in range(NSUB):
            pltpu.async_copy(h_hbm.at[idxs[q].at[j]],
                             rbf[b].at[pl.ds(128 * j, 128)], gsem[b])

    def wait_gathers(b, q):
        for j in range(NSUB):
            pltpu.make_async_copy(h_hbm.at[idxs[q].at[j]],
                                  rbf[b].at[pl.ds(128 * j, 128)],
                                  gsem[b]).wait()

    def scale(b, ob, q):
        for jj in range(2):
            def sgrp(g, carry2):
                wv16 = plsc.bitcast(
                    idxs[q][2 * NSUB + jj, pl.ds(g * 16, 16)], jnp.float32)
                for l in range(16):
                    wv = jnp.full((16,), wv16[l], jnp.float32)
                    k = jj * 128 + g * 16 + l
                    for j2 in range(2):
                        pair = rbf[b][k, pl.ds(32 * j2, 32)]
                        va, vb = plsc.unpack(
                            pair, format=plsc.PackFormat.INTERLEAVED)
                        outs[ob][k, pl.ds(32 * j2, 16)] = va * wv
                        outs[ob][k, pl.ds(32 * j2 + 16, 16)] = vb * wv
                return carry2
            lax.fori_loop(0, 8, sgrp, 0)

    def fire_scatter(ob, q):
        for j in range(NSUB):
            pltpu.async_copy(outs[ob].at[pl.ds(128 * j, 128)],
                             agg_sh.at[idxs[q].at[NSUB + j]], ssem[ob],
                             add=True)

    def wait_scatter(ob, q):
        for j in range(NSUB):
            pltpu.make_async_copy(outs[ob].at[pl.ds(128 * j, 128)],
                                  agg_sh.at[idxs[q].at[NSUB + j]],
                                  ssem[ob]).wait()

    for c in range(4):
        issue_idx(c0 + c, c)
    wait_idx(0)
    fire_gathers(0, 0)
    wait_idx(1)
    fire_gathers(1, 1)

    def body(i, carry):
        for k in range(8):
            c = 8 * i + k
            b = k % 4
            q = k
            ob = k % 2
            bg = (k + 2) % 4
            qg = (k + 2) % 8
            obs = (k + 1) % 2
            qs = (k + 7) % 8
            qi = (k + 4) % 8
            if k < 6:
                wait_idx(qg)
                fire_gathers(bg, qg)
            else:
                @pl.when(i < NI - 1)
                def _():
                    wait_idx(qg)
                    fire_gathers(bg, qg)
            wait_gathers(b, q)
            scale(b, ob, q)
            if k == 0:
                @pl.when(i > 0)
                def _():
                    wait_scatter(obs, qs)
            else:
                wait_scatter(obs, qs)
            fire_scatter(ob, q)
            if k < 4:
                issue_idx(c0 + c + 4, qi)
            else:
                @pl.when(i < NI - 1)
                def _():
                    issue_idx(c0 + c + 4, qi)
        return carry
    NI = CPW // 8
    lax.fori_loop(0, NI, body, 0)
    wait_scatter((CPW - 1) % 2, (CPW - 1) % 8)
    plsc.subcore_barrier()
    pltpu.sync_copy(agg_sh.at[pl.ds(base, N_SHARD)],
                    out_hbm.at[cid, pl.ds(base, N_SHARD)])


_agg_call = functools.partial(
    pl.kernel,
    out_type=jax.ShapeDtypeStruct((2, N_PAD, D), jnp.float32),
    mesh=_mesh,
    scratch_types=(
        [pltpu.VMEM((3 * NSUB, 128), jnp.int32)] * 8
        + [pltpu.VMEM((CHUNK, D), jnp.bfloat16)] * 4
        + [pltpu.VMEM((CHUNK, D), jnp.float32)] * 2
        + [pltpu.VMEM_SHARED((N_PAD, D), jnp.float32)]
        + [pltpu.SemaphoreType.DMA] * 14
    ),
    compiler_params=pltpu.CompilerParams(use_tc_tiling_on_sc=False,
                                         needs_layout_passes=False),
)(_agg_body)


BN = 2000


def _h_body(x_ref, w_ref, o_ref):
    o_ref[...] = jnp.dot(x_ref[...], w_ref[0],
                         preferred_element_type=jnp.float32
                         ).astype(jnp.bfloat16)


def _h_call(x, wrio):
    nb = N // BN
    return pl.pallas_call(
        _h_body,
        grid=(R, nb),
        in_specs=[pl.BlockSpec((BN, D), lambda r, i: (i, 0)),
                  pl.BlockSpec((1, D, D), lambda r, i: (r, 0, 0))],
        out_specs=pl.BlockSpec((BN, D), lambda r, i: (r * nb + i, 0)),
        out_shape=jax.ShapeDtypeStruct((R * N, D), jnp.bfloat16),
    )(x, wrio)


def _comb_body(ap_ref, x_ref, root_ref, b_ref, o_ref):
    acc = (ap_ref[0] + ap_ref[1]
           + jnp.dot(x_ref[...], root_ref[...],
                     preferred_element_type=jnp.float32)
           + b_ref[...])
    o_ref[...] = jnp.maximum(acc, 0.0)


def _comb_call(aggp, x, root, bias):
    return pl.pallas_call(
        _comb_body,
        grid=(N // BN,),
        in_specs=[pl.BlockSpec((2, BN, D), lambda i: (0, i, 0)),
                  pl.BlockSpec((BN, D), lambda i: (i, 0)),
                  pl.BlockSpec((D, D), lambda i: (0, 0)),
                  pl.BlockSpec((1, D), lambda i: (0, 0))],
        out_specs=pl.BlockSpec((BN, D), lambda i: (i, 0)),
        out_shape=jax.ShapeDtypeStruct((N, D), jnp.float32),
    )(aggp, x, root, bias.reshape(1, D))


def kernel(ent_emb, comp0, bases0, root0, bias0, comp1, bases1, root1, bias1,
           edge_index, edge_type):
    src = edge_index[0].astype(jnp.int32)
    dst = edge_index[1].astype(jnp.int32)
    et = edge_type.astype(jnp.int32)
    pad = E_PAD - E
    trash_pair = NR + (jnp.arange(pad, dtype=jnp.int32) % (NR_PAD - NR))
    trash_dst = N + (jnp.arange(pad, dtype=jnp.int32) % (N_PAD - N))
    gidx = jnp.concatenate([et * N + src,
                            jnp.zeros((pad,), jnp.int32)]).reshape(
                                NCHUNK, NSUB, 128)
    npair = jnp.concatenate([dst * R + et,
                             trash_pair]).reshape(NCHUNK, NSUB, 128)
    dsti = jnp.concatenate([dst, trash_dst]).reshape(NCHUNK, NSUB, 128)

    cntp = _cnt_call(npair)
    norm = 1.0 / jnp.maximum(cntp[0] + cntp[1], 1.0)
    wedge = jnp.take(norm, npair.reshape(-1), axis=0)
    wbits = lax.bitcast_convert_type(wedge, jnp.int32).reshape(
        NCHUNK, NSUB, 128)
    idxall = jnp.concatenate([gidx, dsti, wbits], axis=1)

    x = ent_emb
    for comp, bases, root, bias in ((comp0, bases0, root0, bias0),
                                    (comp1, bases1, root1, bias1)):
        wrio = jnp.einsum('rb,bio->rio', comp, bases)[:, :, _P64]
        h = _h_call(x, wrio)
        aggp = _agg_call(idxall, h)
        x = _comb_call(aggp, x, root, bias)
    return x


# speedup vs baseline: 6.0662x; 6.0662x over previous
import functools

import numpy as np

import jax
import jax.numpy as jnp
from jax import lax
from jax.experimental import pallas as pl
from jax.experimental.pallas import tpu as pltpu
from jax.experimental.pallas import tpu_sc as plsc

N = 10000
R = 20
D = 64
E = 640000

NR = N * R
NR_PAD = 200704
NR_SHARD = NR_PAD // 16
N_PAD = 10112
N_SHARD = N_PAD // 16

CHUNK = 256
NSUB = CHUNK // 128
NCHUNK = 2560
E_PAD = NCHUNK * CHUNK
NW = 32
CPW = NCHUNK // NW

_p32 = np.empty((32,), np.int32)
_p32[0::2] = np.arange(16)
_p32[1::2] = 16 + np.arange(16)
_P64 = np.concatenate([_p32, 32 + _p32])

_mesh = plsc.VectorSubcoreMesh(core_axis_name="c", subcore_axis_name="s",
                               num_cores=2, num_subcores=16)


def _cnt_body(npair_hbm, out_hbm, idx_v, ones_v, zst_v, cnt_sh, sem):
    cid = lax.axis_index("c")
    sid = lax.axis_index("s")
    wid = sid * 2 + cid

    def zb(i, carry):
        zst_v[pl.ds(i * 16, 16)] = jnp.zeros((16,), jnp.float32)
        return carry
    lax.fori_loop(0, NR_SHARD // 16, zb, 0)
    for j in range(8):
        ones_v[pl.ds(16 * j, 16)] = jnp.ones((16,), jnp.float32)
    pltpu.sync_copy(zst_v, cnt_sh.at[pl.ds(sid * NR_SHARD, NR_SHARD)])
    plsc.subcore_barrier()

    def cb(i, carry):
        c = wid * CPW + i
        pltpu.sync_copy(npair_hbm.at[c], idx_v)
        for j in range(NSUB):
            pltpu.sync_copy(ones_v, cnt_sh.at[idx_v.at[j]], add=True)
        return carry
    lax.fori_loop(0, CPW, cb, 0)
    plsc.subcore_barrier()
    pltpu.sync_copy(cnt_sh.at[pl.ds(sid * NR_SHARD, NR_SHARD)],
                    out_hbm.at[cid, pl.ds(sid * NR_SHARD, NR_SHARD)])


_cnt_call = functools.partial(
    pl.kernel,
    out_type=jax.ShapeDtypeStruct((2, NR_PAD), jnp.float32),
    mesh=_mesh,
    scratch_types=[
        pltpu.VMEM((NSUB, 128), jnp.int32),
        pltpu.VMEM((128,), jnp.float32),
        pltpu.VMEM((NR_SHARD,), jnp.float32),
        pltpu.VMEM_SHARED((NR_PAD,), jnp.float32),
        pltpu.SemaphoreType.DMA,
    ],
)(_cnt_body)


def _agg_body(idxall_hbm, h_hbm, out_hbm,
              i0, i1, i2, i3, i4, i5, i6, i7,
              rb0, rb1, rb2, rb3, o0, o1, agg_sh,
              is0, is1, is2, is3, is4, is5, is6, is7,
              gs0, gs1, gs2, gs3, ss0, ss1):
    cid = lax.axis_index("c")
    sid = lax.axis_index("s")
    wid = sid * 2 + cid
    idxs = (i0, i1, i2, i3, i4, i5, i6, i7)
    rbf = (rb0, rb1, rb2, rb3)
    outs = (o0, o1)
    isem = (is0, is1, is2, is3, is4, is5, is6, is7)
    gsem = (gs0, gs1, gs2, gs3)
    ssem = (ss0, ss1)
    c0 = wid * CPW

    def zr(k, carry):
        for j in range(4):
            o0[k, pl.ds(16 * j, 16)] = jnp.zeros((16,), jnp.float32)
        return carry
    lax.fori_loop(0, CHUNK, zr, 0)
    base = sid * N_SHARD
    pltpu.sync_copy(o0, agg_sh.at[pl.ds(base, CHUNK)])
    pltpu.sync_copy(o0, agg_sh.at[pl.ds(base + CHUNK, CHUNK)])
    pltpu.sync_copy(o0.at[pl.ds(0, N_SHARD - 2 * CHUNK)],
                    agg_sh.at[pl.ds(base + 2 * CHUNK, N_SHARD - 2 * CHUNK)])
    plsc.subcore_barrier()

    def issue_idx(c, q):
        pltpu.async_copy(idxall_hbm.at[c], idxs[q], isem[q])

    def wait_idx(q):
        pltpu.make_async_copy(idxall_hbm.at[0], idxs[q], isem[q]).wait()

    def fire_gathers(b, q):
        for j in range(NSUB):
            pltpu.async_copy(h_hbm.at[idxs[q].at[j]],
                             rbf[b].at[pl.ds(128 * j, 128)], gsem[b])

    def wait_gathers(b, q):
        for j in range(NSUB):
            pltpu.make_async_copy(h_hbm.at[idxs[q].at[j]],
                                  rbf[b].at[pl.ds(128 * j, 128)],
                                  gsem[b]).wait()

    def scale(b, ob, q):
        for jj in range(2):
            def sgrp(g, carry2):
                wv16 = plsc.bitcast(
                    idxs[q][2 * NSUB + jj, pl.ds(g * 16, 16)], jnp.float32)
                for l in range(16):
                    wv = jnp.full((16,), wv16[l], jnp.float32)
                    k = jj * 128 + g * 16 + l
                    for j2 in range(2):
                        pair = rbf[b][k, pl.ds(32 * j2, 32)]
                        va, vb = plsc.unpack(
                            pair, format=plsc.PackFormat.INTERLEAVED)
                        outs[ob][k, pl.ds(32 * j2, 16)] = va * wv
                        outs[ob][k, pl.ds(32 * j2 + 16, 16)] = vb * wv
                return carry2
            lax.fori_loop(0, 8, sgrp, 0)

    def fire_scatter(ob, q):
        for j in range(NSUB):
            pltpu.async_copy(outs[ob].at[pl.ds(128 * j, 128)],
                             agg_sh.at[idxs[q].at[NSUB + j]], ssem[ob],
                             add=True)

    def wait_scatter(ob, q):
        for j in range(NSUB):
            pltpu.make_async_copy(outs[ob].at[pl.ds(128 * j, 128)],
                                  agg_sh.at[idxs[q].at[NSUB + j]],
                                  ssem[ob]).wait()

    for c in range(4):
        issue_idx(c0 + c, c)
    wait_idx(0)
    fire_gathers(0, 0)
    wait_idx(1)
    fire_gathers(1, 1)

    def body(i, carry):
        for k in range(8):
            c = 8 * i + k
            b = k % 4
            q = k
            ob = k % 2
            bg = (k + 2) % 4
            qg = (k + 2) % 8
            obs = (k + 1) % 2
            qs = (k + 7) % 8
            qi = (k + 4) % 8
            if k < 6:
                wait_idx(qg)
                fire_gathers(bg, qg)
            else:
                @pl.when(i < NI - 1)
                def _():
                    wait_idx(qg)
                    fire_gathers(bg, qg)
            wait_gathers(b, q)
            scale(b, ob, q)
            if k == 0:
                @pl.when(i > 0)
                def _():
                    wait_scatter(obs, qs)
            else:
                wait_scatter(obs, qs)
            fire_scatter(ob, q)
            if k < 4:
                issue_idx(c0 + c + 4, qi)
            else:
                @pl.when(i < NI - 1)
                def _():
                    issue_idx(c0 + c + 4, qi)
        return carry
    NI = CPW // 8
    lax.fori_loop(0, NI, body, 0)
    wait_scatter((CPW - 1) % 2, (CPW - 1) % 8)
    plsc.subcore_barrier()
    pltpu.sync_copy(agg_sh.at[pl.ds(base, N_SHARD)],
                    out_hbm.at[cid, pl.ds(base, N_SHARD)])


_agg_call = functools.partial(
    pl.kernel,
    out_type=jax.ShapeDtypeStruct((2, N_PAD, D), jnp.float32),
    mesh=_mesh,
    scratch_types=(
        [pltpu.VMEM((3 * NSUB, 128), jnp.int32)] * 8
        + [pltpu.VMEM((CHUNK, D), jnp.bfloat16)] * 4
        + [pltpu.VMEM((CHUNK, D), jnp.float32)] * 2
        + [pltpu.VMEM_SHARED((N_PAD, D), jnp.float32)]
        + [pltpu.SemaphoreType.DMA] * 14
    ),
    compiler_params=pltpu.CompilerParams(use_tc_tiling_on_sc=False,
                                         needs_layout_passes=False),
)(_agg_body)


def _wg_body(npair_hbm, norm_hbm, wout_hbm,
             ib0, ib1, wb0, wb1, is0, is1, gs0, gs1, ws0, ws1):
    cid = lax.axis_index("c")
    sid = lax.axis_index("s")
    wid = sid * 2 + cid
    ibuf = (ib0, ib1)
    wbuf = (wb0, wb1)
    isem = (is0, is1)
    gsem = (gs0, gs1)
    wsem = (ws0, ws1)
    c0 = wid * CPW

    def issue_idx(c, b):
        pltpu.async_copy(npair_hbm.at[c], ibuf[b], isem[b])

    def wait_idx(b):
        pltpu.make_async_copy(npair_hbm.at[0], ibuf[b], isem[b]).wait()

    def fire_gathers(b):
        for j in range(NSUB):
            pltpu.async_copy(norm_hbm.at[ibuf[b].at[j]],
                             wbuf[b].at[j], gsem[b])

    def wait_gathers(b):
        for j in range(NSUB):
            pltpu.make_async_copy(norm_hbm.at[ibuf[b].at[j]],
                                  wbuf[b].at[j], gsem[b]).wait()

    def fire_write(c, b):
        pltpu.async_copy(wbuf[b], wout_hbm.at[c], wsem[b])

    def wait_write(b):
        pltpu.make_async_copy(wbuf[b], wout_hbm.at[0], wsem[b]).wait()

    issue_idx(c0, 0)
    wait_idx(0)
    fire_gathers(0)

    def body(i, carry):
        for k in range(2):
            c = 2 * i + k
            b = k
            o = 1 - k
            if k == 0:
                issue_idx(c0 + c + 1, o)
            else:
                @pl.when(i < CPW // 2 - 1)
                def _():
                    issue_idx(c0 + c + 1, o)
            wait_gathers(b)
            fire_write(c0 + c, b)
            if k == 0:
                @pl.when(i > 0)
                def _():
                    wait_write(o)

                def _fg():
                    wait_idx(o)
                    fire_gathers(o)
                _fg()
            else:
                @pl.when(i < CPW // 2 - 1)
                def _():
                    wait_write(o)
                    wait_idx(o)
                    fire_gathers(o)
        return carry
    lax.fori_loop(0, CPW // 2, body, 0)
    wait_write(0)
    wait_write(1)
    plsc.subcore_barrier()


_wg_call = functools.partial(
    pl.kernel,
    out_type=jax.ShapeDtypeStruct((NCHUNK, NSUB, 128), jnp.float32),
    mesh=_mesh,
    scratch_types=(
        [pltpu.VMEM((NSUB, 128), jnp.int32)] * 2
        + [pltpu.VMEM((NSUB, 128), jnp.float32)] * 2
        + [pltpu.SemaphoreType.DMA] * 6
    ),
    compiler_params=pltpu.CompilerParams(use_tc_tiling_on_sc=False,
                                         needs_layout_passes=False),
)(_wg_body)


BN = 2000


def _h_body(x_ref, w_ref, o_ref):
    o_ref[...] = jnp.dot(x_ref[...], w_ref[0],
                         preferred_element_type=jnp.float32
                         ).astype(jnp.bfloat16)


def _h_call(x, wrio):
    nb = N // BN
    return pl.pallas_call(
        _h_body,
        grid=(R, nb),
        in_specs=[pl.BlockSpec((BN, D), lambda r, i: (i, 0)),
                  pl.BlockSpec((1, D, D), lambda r, i: (r, 0, 0))],
        out_specs=pl.BlockSpec((BN, D), lambda r, i: (r * nb + i, 0)),
        out_shape=jax.ShapeDtypeStruct((R * N, D), jnp.bfloat16),
    )(x, wrio)


def _comb_body(ap_ref, x_ref, root_ref, b_ref, o_ref):
    acc = (ap_ref[0] + ap_ref[1]
           + jnp.dot(x_ref[...], root_ref[...],
                     preferred_element_type=jnp.float32)
           + b_ref[...])
    o_ref[...] = jnp.maximum(acc, 0.0)


def _comb_call(aggp, x, root, bias):
    return pl.pallas_call(
        _comb_body,
        grid=(N // BN,),
        in_specs=[pl.BlockSpec((2, BN, D), lambda i: (0, i, 0)),
                  pl.BlockSpec((BN, D), lambda i: (i, 0)),
                  pl.BlockSpec((D, D), lambda i: (0, 0)),
                  pl.BlockSpec((1, D), lambda i: (0, 0))],
        out_specs=pl.BlockSpec((BN, D), lambda i: (i, 0)),
        out_shape=jax.ShapeDtypeStruct((N, D), jnp.float32),
    )(aggp, x, root, bias.reshape(1, D))


def kernel(ent_emb, comp0, bases0, root0, bias0, comp1, bases1, root1, bias1,
           edge_index, edge_type):
    src = edge_index[0].astype(jnp.int32)
    dst = edge_index[1].astype(jnp.int32)
    et = edge_type.astype(jnp.int32)
    pad = E_PAD - E
    trash_pair = NR + (jnp.arange(pad, dtype=jnp.int32) % (NR_PAD - NR))
    trash_dst = N + (jnp.arange(pad, dtype=jnp.int32) % (N_PAD - N))
    gidx = jnp.concatenate([et * N + src,
                            jnp.zeros((pad,), jnp.int32)]).reshape(
                                NCHUNK, NSUB, 128)
    npair = jnp.concatenate([dst * R + et,
                             trash_pair]).reshape(NCHUNK, NSUB, 128)
    dsti = jnp.concatenate([dst, trash_dst]).reshape(NCHUNK, NSUB, 128)

    cntp = _cnt_call(npair)
    norm = 1.0 / jnp.maximum(cntp[0] + cntp[1], 1.0)
    wedge = _wg_call(npair, norm)
    wbits = lax.bitcast_convert_type(wedge, jnp.int32)
    idxall = jnp.concatenate([gidx, dsti, wbits], axis=1)

    x = ent_emb
    for comp, bases, root, bias in ((comp0, bases0, root0, bias0),
                                    (comp1, bases1, root1, bias1)):
        wrio = jnp.einsum('rb,bio->rio', comp, bases)[:, :, _P64]
        h = _h_call(x, wrio)
        aggp = _agg_call(idxall, h)
        x = _comb_call(aggp, x, root, bias)
    return x


# speedup vs baseline: 7.1892x; 1.1851x over previous
import functools

import numpy as np

import jax
import jax.numpy as jnp
from jax import lax
from jax.experimental import pallas as pl
from jax.experimental.pallas import tpu as pltpu
from jax.experimental.pallas import tpu_sc as plsc

N = 10000
R = 20
D = 64
E = 640000

NR = N * R
NR_PAD = 200704
NR_SHARD = NR_PAD // 16
N_PAD = 10112
N_SHARD = N_PAD // 16

CHUNK = 256
NSUB = CHUNK // 128
NCHUNK = 2560
E_PAD = NCHUNK * CHUNK
NW = 32
CPW = NCHUNK // NW

_p32 = np.empty((32,), np.int32)
_p32[0::2] = np.arange(16)
_p32[1::2] = 16 + np.arange(16)
_P64 = np.concatenate([_p32, 32 + _p32])

_mesh = plsc.VectorSubcoreMesh(core_axis_name="c", subcore_axis_name="s",
                               num_cores=2, num_subcores=16)


def _cnt_body(npair_hbm, out_hbm, idx_v, ones_v, zst_v, cnt_sh, sem):
    cid = lax.axis_index("c")
    sid = lax.axis_index("s")
    wid = sid * 2 + cid

    def zb(i, carry):
        zst_v[pl.ds(i * 16, 16)] = jnp.zeros((16,), jnp.float32)
        return carry
    lax.fori_loop(0, NR_SHARD // 16, zb, 0)
    for j in range(8):
        ones_v[pl.ds(16 * j, 16)] = jnp.ones((16,), jnp.float32)
    pltpu.sync_copy(zst_v, cnt_sh.at[pl.ds(sid * NR_SHARD, NR_SHARD)])
    plsc.subcore_barrier()

    def cb(i, carry):
        c = wid * CPW + i
        pltpu.sync_copy(npair_hbm.at[c], idx_v)
        for j in range(NSUB):
            pltpu.sync_copy(ones_v, cnt_sh.at[idx_v.at[j]], add=True)
        return carry
    lax.fori_loop(0, CPW, cb, 0)
    plsc.subcore_barrier()
    pltpu.sync_copy(cnt_sh.at[pl.ds(sid * NR_SHARD, NR_SHARD)],
                    out_hbm.at[cid, pl.ds(sid * NR_SHARD, NR_SHARD)])


_cnt_call = functools.partial(
    pl.kernel,
    out_type=jax.ShapeDtypeStruct((2, NR_PAD), jnp.float32),
    mesh=_mesh,
    scratch_types=[
        pltpu.VMEM((NSUB, 128), jnp.int32),
        pltpu.VMEM((128,), jnp.float32),
        pltpu.VMEM((NR_SHARD,), jnp.float32),
        pltpu.VMEM_SHARED((NR_PAD,), jnp.float32),
        pltpu.SemaphoreType.DMA,
    ],
)(_cnt_body)


def _agg_body(idxall_hbm, h_hbm, norm_hbm, out_hbm,
              i0, i1, i2, i3, i4, i5, i6, i7,
              w0, w1, w2, w3, rb0, rb1, rb2, rb3, o0, o1, agg_sh,
              is0, is1, is2, is3, is4, is5, is6, is7,
              gs0, gs1, gs2, gs3, ss0, ss1):
    cid = lax.axis_index("c")
    sid = lax.axis_index("s")
    wid = sid * 2 + cid
    idxs = (i0, i1, i2, i3, i4, i5, i6, i7)
    ws = (w0, w1, w2, w3)
    rbf = (rb0, rb1, rb2, rb3)
    outs = (o0, o1)
    isem = (is0, is1, is2, is3, is4, is5, is6, is7)
    gsem = (gs0, gs1, gs2, gs3)
    ssem = (ss0, ss1)
    c0 = wid * CPW

    def zr(k, carry):
        for j in range(4):
            o0[k, pl.ds(16 * j, 16)] = jnp.zeros((16,), jnp.float32)
        return carry
    lax.fori_loop(0, CHUNK, zr, 0)
    base = sid * N_SHARD
    pltpu.sync_copy(o0, agg_sh.at[pl.ds(base, CHUNK)])
    pltpu.sync_copy(o0, agg_sh.at[pl.ds(base + CHUNK, CHUNK)])
    pltpu.sync_copy(o0.at[pl.ds(0, N_SHARD - 2 * CHUNK)],
                    agg_sh.at[pl.ds(base + 2 * CHUNK, N_SHARD - 2 * CHUNK)])
    plsc.subcore_barrier()

    def issue_idx(c, q):
        pltpu.async_copy(idxall_hbm.at[c], idxs[q], isem[q])

    def wait_idx(q):
        pltpu.make_async_copy(idxall_hbm.at[0], idxs[q], isem[q]).wait()

    def fire_gathers(b, q):
        for j in range(NSUB):
            pltpu.async_copy(h_hbm.at[idxs[q].at[j]],
                             rbf[b].at[pl.ds(128 * j, 128)], gsem[b])
        for j in range(NSUB):
            pltpu.async_copy(norm_hbm.at[idxs[q].at[NSUB + j]],
                             ws[b].at[pl.ds(128 * j, 128)], gsem[b])

    def wait_gathers(b, q):
        for j in range(NSUB):
            pltpu.make_async_copy(h_hbm.at[idxs[q].at[j]],
                                  rbf[b].at[pl.ds(128 * j, 128)],
                                  gsem[b]).wait()
        for j in range(NSUB):
            pltpu.make_async_copy(norm_hbm.at[idxs[q].at[NSUB + j]],
                                  ws[b].at[pl.ds(128 * j, 128)],
                                  gsem[b]).wait()

    def scale(b, ob):
        def sgrp(g, carry2):
            wv16 = ws[b][pl.ds(g * 16, 16)]
            for l in range(16):
                wv = jnp.full((16,), wv16[l], jnp.float32)
                k = g * 16 + l
                for j2 in range(2):
                    pair = rbf[b][k, pl.ds(32 * j2, 32)]
                    va, vb = plsc.unpack(
                        pair, format=plsc.PackFormat.INTERLEAVED)
                    outs[ob][k, pl.ds(32 * j2, 16)] = va * wv
                    outs[ob][k, pl.ds(32 * j2 + 16, 16)] = vb * wv
            return carry2
        lax.fori_loop(0, CHUNK // 16, sgrp, 0)

    def fire_scatter(ob, q):
        for j in range(NSUB):
            pltpu.async_copy(outs[ob].at[pl.ds(128 * j, 128)],
                             agg_sh.at[idxs[q].at[2 * NSUB + j]], ssem[ob],
                             add=True)

    def wait_scatter(ob, q):
        for j in range(NSUB):
            pltpu.make_async_copy(outs[ob].at[pl.ds(128 * j, 128)],
                                  agg_sh.at[idxs[q].at[2 * NSUB + j]],
                                  ssem[ob]).wait()

    for c in range(5):
        issue_idx(c0 + c, c)
    for c in range(3):
        wait_idx(c)
        fire_gathers(c, c)

    def body(i, carry):
        for k in range(8):
            c = 8 * i + k
            b = k % 4
            q = k
            ob = k % 2
            bg = (k + 3) % 4
            qg = (k + 3) % 8
            obs = (k + 1) % 2
            qs = (k + 7) % 8
            qi = (k + 5) % 8
            if k < 5:
                wait_idx(qg)
                fire_gathers(bg, qg)
            else:
                @pl.when(i < NI - 1)
                def _():
                    wait_idx(qg)
                    fire_gathers(bg, qg)
            wait_gathers(b, q)
            scale(b, ob)
            if k == 0:
                @pl.when(i > 0)
                def _():
                    wait_scatter(obs, qs)
            else:
                wait_scatter(obs, qs)
            fire_scatter(ob, q)
            if k < 3:
                issue_idx(c0 + c + 5, qi)
            else:
                @pl.when(i < NI - 1)
                def _():
                    issue_idx(c0 + c + 5, qi)
        return carry
    NI = CPW // 8
    lax.fori_loop(0, NI, body, 0)
    wait_scatter((CPW - 1) % 2, (CPW - 1) % 8)
    plsc.subcore_barrier()
    pltpu.sync_copy(agg_sh.at[pl.ds(base, N_SHARD)],
                    out_hbm.at[cid, pl.ds(base, N_SHARD)])


_agg_call = functools.partial(
    pl.kernel,
    out_type=jax.ShapeDtypeStruct((2, N_PAD, D), jnp.float32),
    mesh=_mesh,
    scratch_types=(
        [pltpu.VMEM((3 * NSUB, 128), jnp.int32)] * 8
        + [pltpu.VMEM((CHUNK,), jnp.float32)] * 4
        + [pltpu.VMEM((CHUNK, D), jnp.bfloat16)] * 4
        + [pltpu.VMEM((CHUNK, D), jnp.float32)] * 2
        + [pltpu.VMEM_SHARED((N_PAD, D), jnp.float32)]
        + [pltpu.SemaphoreType.DMA] * 14
    ),
    compiler_params=pltpu.CompilerParams(use_tc_tiling_on_sc=False,
                                         needs_layout_passes=False),
)(_agg_body)


BN = 2000


def _h_body(x_ref, w_ref, o_ref):
    o_ref[...] = jnp.dot(x_ref[...], w_ref[0],
                         preferred_element_type=jnp.float32
                         ).astype(jnp.bfloat16)


def _h_call(x, wrio):
    nb = N // BN
    return pl.pallas_call(
        _h_body,
        grid=(R, nb),
        in_specs=[pl.BlockSpec((BN, D), lambda r, i: (i, 0)),
                  pl.BlockSpec((1, D, D), lambda r, i: (r, 0, 0))],
        out_specs=pl.BlockSpec((BN, D), lambda r, i: (r * nb + i, 0)),
        out_shape=jax.ShapeDtypeStruct((R * N, D), jnp.bfloat16),
    )(x, wrio)


def _comb_body(ap_ref, x_ref, root_ref, b_ref, o_ref):
    acc = (ap_ref[0] + ap_ref[1]
           + jnp.dot(x_ref[...], root_ref[...],
                     preferred_element_type=jnp.float32)
           + b_ref[...])
    o_ref[...] = jnp.maximum(acc, 0.0)


def _comb_call(aggp, x, root, bias):
    return pl.pallas_call(
        _comb_body,
        grid=(N // BN,),
        in_specs=[pl.BlockSpec((2, BN, D), lambda i: (0, i, 0)),
                  pl.BlockSpec((BN, D), lambda i: (i, 0)),
                  pl.BlockSpec((D, D), lambda i: (0, 0)),
                  pl.BlockSpec((1, D), lambda i: (0, 0))],
        out_specs=pl.BlockSpec((BN, D), lambda i: (i, 0)),
        out_shape=jax.ShapeDtypeStruct((N, D), jnp.float32),
    )(aggp, x, root, bias.reshape(1, D))


def kernel(ent_emb, comp0, bases0, root0, bias0, comp1, bases1, root1, bias1,
           edge_index, edge_type):
    src = edge_index[0].astype(jnp.int32)
    dst = edge_index[1].astype(jnp.int32)
    et = edge_type.astype(jnp.int32)
    pad = E_PAD - E
    trash_pair = NR + (jnp.arange(pad, dtype=jnp.int32) % (NR_PAD - NR))
    trash_dst = N + (jnp.arange(pad, dtype=jnp.int32) % (N_PAD - N))
    gidx = jnp.concatenate([et * N + src,
                            jnp.zeros((pad,), jnp.int32)]).reshape(
                                NCHUNK, NSUB, 128)
    npair = jnp.concatenate([dst * R + et,
                             trash_pair]).reshape(NCHUNK, NSUB, 128)
    dsti = jnp.concatenate([dst, trash_dst]).reshape(NCHUNK, NSUB, 128)

    idxall = jnp.concatenate([gidx, npair, dsti], axis=1)

    cntp = _cnt_call(npair)
    norm = 1.0 / jnp.maximum(cntp[0] + cntp[1], 1.0)

    x = ent_emb
    for comp, bases, root, bias in ((comp0, bases0, root0, bias0),
                                    (comp1, bases1, root1, bias1)):
        wrio = jnp.einsum('rb,bio->rio', comp, bases)[:, :, _P64]
        h = _h_call(x, wrio)
        aggp = _agg_call(idxall, h, norm)
        x = _comb_call(aggp, x, root, bias)
    return x


# speedup vs baseline: 8.8555x; 1.2318x over previous
import functools

import numpy as np

import jax
import jax.numpy as jnp
from jax import lax
from jax.experimental import pallas as pl
from jax.experimental.pallas import tpu as pltpu
from jax.experimental.pallas import tpu_sc as plsc

N = 10000
R = 20
D = 64
E = 640000

NR = N * R
NR_PAD = 200704
NR_SHARD = NR_PAD // 16
N_PAD = 10112
N_SHARD = N_PAD // 16

CHUNK = 256
NSUB = CHUNK // 128
NCHUNK = 2560
E_PAD = NCHUNK * CHUNK
NW = 32
CPW = NCHUNK // NW

_p32 = np.empty((32,), np.int32)
_p32[0::2] = np.arange(16)
_p32[1::2] = 16 + np.arange(16)
_HPERM = np.concatenate([r * 64 + blk + _p32
                         for r in range(R) for blk in (0, 32)])

_mesh = plsc.VectorSubcoreMesh(core_axis_name="c", subcore_axis_name="s",
                               num_cores=2, num_subcores=16)


def _cnt_body(npair_hbm, out_hbm, idx_v, ones_v, zst_v, cnt_sh, sem):
    cid = lax.axis_index("c")
    sid = lax.axis_index("s")
    wid = sid * 2 + cid

    def zb(i, carry):
        zst_v[pl.ds(i * 16, 16)] = jnp.zeros((16,), jnp.float32)
        return carry
    lax.fori_loop(0, NR_SHARD // 16, zb, 0)
    for j in range(8):
        ones_v[pl.ds(16 * j, 16)] = jnp.ones((16,), jnp.float32)
    pltpu.sync_copy(zst_v, cnt_sh.at[pl.ds(sid * NR_SHARD, NR_SHARD)])
    plsc.subcore_barrier()

    def cb(i, carry):
        c = wid * CPW + i
        pltpu.sync_copy(npair_hbm.at[c], idx_v)
        for j in range(NSUB):
            pltpu.sync_copy(ones_v, cnt_sh.at[idx_v.at[j]], add=True)
        return carry
    lax.fori_loop(0, CPW, cb, 0)
    plsc.subcore_barrier()
    pltpu.sync_copy(cnt_sh.at[pl.ds(sid * NR_SHARD, NR_SHARD)],
                    out_hbm.at[cid, pl.ds(sid * NR_SHARD, NR_SHARD)])


_cnt_call = functools.partial(
    pl.kernel,
    out_type=jax.ShapeDtypeStruct((2, NR_PAD), jnp.float32),
    mesh=_mesh,
    scratch_types=[
        pltpu.VMEM((NSUB, 128), jnp.int32),
        pltpu.VMEM((128,), jnp.float32),
        pltpu.VMEM((NR_SHARD,), jnp.float32),
        pltpu.VMEM_SHARED((NR_PAD,), jnp.float32),
        pltpu.SemaphoreType.DMA,
    ],
)(_cnt_body)


def _agg_body(idxall_hbm, h_hbm, norm_hbm, out_hbm,
              i0, i1, i2, i3, i4, i5, i6, i7,
              w0, w1, w2, w3, rb0, rb1, rb2, rb3, o0, o1, agg_sh,
              is0, is1, is2, is3, is4, is5, is6, is7,
              gs0, gs1, gs2, gs3, ss0, ss1):
    cid = lax.axis_index("c")
    sid = lax.axis_index("s")
    wid = sid * 2 + cid
    idxs = (i0, i1, i2, i3, i4, i5, i6, i7)
    ws = (w0, w1, w2, w3)
    rbf = (rb0, rb1, rb2, rb3)
    outs = (o0, o1)
    isem = (is0, is1, is2, is3, is4, is5, is6, is7)
    gsem = (gs0, gs1, gs2, gs3)
    ssem = (ss0, ss1)
    c0 = wid * CPW

    def zr(k, carry):
        for j in range(4):
            o0[k, pl.ds(16 * j, 16)] = jnp.zeros((16,), jnp.float32)
        return carry
    lax.fori_loop(0, CHUNK, zr, 0)
    base = sid * N_SHARD
    pltpu.sync_copy(o0, agg_sh.at[pl.ds(base, CHUNK)])
    pltpu.sync_copy(o0, agg_sh.at[pl.ds(base + CHUNK, CHUNK)])
    pltpu.sync_copy(o0.at[pl.ds(0, N_SHARD - 2 * CHUNK)],
                    agg_sh.at[pl.ds(base + 2 * CHUNK, N_SHARD - 2 * CHUNK)])
    plsc.subcore_barrier()

    def issue_idx(c, q):
        pltpu.async_copy(idxall_hbm.at[c], idxs[q], isem[q])

    def wait_idx(q):
        pltpu.make_async_copy(idxall_hbm.at[0], idxs[q], isem[q]).wait()

    def fire_gathers(b, q):
        for j in range(NSUB):
            pltpu.async_copy(h_hbm.at[idxs[q].at[j]],
                             rbf[b].at[pl.ds(128 * j, 128)], gsem[b])
        for j in range(NSUB):
            pltpu.async_copy(norm_hbm.at[idxs[q].at[NSUB + j]],
                             ws[b].at[pl.ds(128 * j, 128)], gsem[b])

    def wait_gathers(b, q):
        for j in range(NSUB):
            pltpu.make_async_copy(h_hbm.at[idxs[q].at[j]],
                                  rbf[b].at[pl.ds(128 * j, 128)],
                                  gsem[b]).wait()
        for j in range(NSUB):
            pltpu.make_async_copy(norm_hbm.at[idxs[q].at[NSUB + j]],
                                  ws[b].at[pl.ds(128 * j, 128)],
                                  gsem[b]).wait()

    def scale(b, ob):
        def sgrp(g, carry2):
            wv16 = ws[b][pl.ds(g * 16, 16)]
            for l in range(16):
                wv = jnp.full((16,), wv16[l], jnp.float32)
                k = g * 16 + l
                for j2 in range(2):
                    pair = rbf[b][k, pl.ds(32 * j2, 32)]
                    va, vb = plsc.unpack(
                        pair, format=plsc.PackFormat.INTERLEAVED)
                    outs[ob][k, pl.ds(32 * j2, 16)] = va * wv
                    outs[ob][k, pl.ds(32 * j2 + 16, 16)] = vb * wv
            return carry2
        lax.fori_loop(0, CHUNK // 16, sgrp, 0)

    def fire_scatter(ob, q):
        for j in range(NSUB):
            pltpu.async_copy(outs[ob].at[pl.ds(128 * j, 128)],
                             agg_sh.at[idxs[q].at[2 * NSUB + j]], ssem[ob],
                             add=True)

    def wait_scatter(ob, q):
        for j in range(NSUB):
            pltpu.make_async_copy(outs[ob].at[pl.ds(128 * j, 128)],
                                  agg_sh.at[idxs[q].at[2 * NSUB + j]],
                                  ssem[ob]).wait()

    for c in range(5):
        issue_idx(c0 + c, c)
    for c in range(3):
        wait_idx(c)
        fire_gathers(c, c)

    def body(i, carry):
        for k in range(8):
            c = 8 * i + k
            b = k % 4
            q = k
            ob = k % 2
            bg = (k + 3) % 4
            qg = (k + 3) % 8
            obs = (k + 1) % 2
            qs = (k + 7) % 8
            qi = (k + 5) % 8
            if k < 5:
                wait_idx(qg)
                fire_gathers(bg, qg)
            else:
                @pl.when(i < NI - 1)
                def _():
                    wait_idx(qg)
                    fire_gathers(bg, qg)
            wait_gathers(b, q)
            scale(b, ob)
            if k == 0:
                @pl.when(i > 0)
                def _():
                    wait_scatter(obs, qs)
            else:
                wait_scatter(obs, qs)
            fire_scatter(ob, q)
            if k < 3:
                issue_idx(c0 + c + 5, qi)
            else:
                @pl.when(i < NI - 1)
                def _():
                    issue_idx(c0 + c + 5, qi)
        return carry
    NI = CPW // 8
    lax.fori_loop(0, NI, body, 0)
    wait_scatter((CPW - 1) % 2, (CPW - 1) % 8)
    plsc.subcore_barrier()
    pltpu.sync_copy(agg_sh.at[pl.ds(base, N_SHARD)],
                    out_hbm.at[cid, pl.ds(base, N_SHARD)])


_agg_call = functools.partial(
    pl.kernel,
    out_type=jax.ShapeDtypeStruct((2, N_PAD, D), jnp.float32),
    mesh=_mesh,
    scratch_types=(
        [pltpu.VMEM((3 * NSUB, 128), jnp.int32)] * 8
        + [pltpu.VMEM((CHUNK,), jnp.float32)] * 4
        + [pltpu.VMEM((CHUNK, D), jnp.bfloat16)] * 4
        + [pltpu.VMEM((CHUNK, D), jnp.float32)] * 2
        + [pltpu.VMEM_SHARED((N_PAD, D), jnp.float32)]
        + [pltpu.SemaphoreType.DMA] * 14
    ),
    compiler_params=pltpu.CompilerParams(use_tc_tiling_on_sc=False,
                                         needs_layout_passes=False),
)(_agg_body)


BN = 2000


def _h_body(x_ref, w_ref, o_ref):
    o_ref[...] = jnp.dot(x_ref[...], w_ref[...],
                         preferred_element_type=jnp.float32
                         ).astype(jnp.bfloat16)


def _h_call(x, wcat):
    return pl.pallas_call(
        _h_body,
        grid=(N // BN,),
        in_specs=[pl.BlockSpec((BN, D), lambda i: (i, 0)),
                  pl.BlockSpec((D, R * D), lambda i: (0, 0))],
        out_specs=pl.BlockSpec((BN, R * D), lambda i: (i, 0)),
        out_shape=jax.ShapeDtypeStruct((N, R * D), jnp.bfloat16),
    )(x, wcat)


def _comb_body(ap_ref, x_ref, root_ref, b_ref, o_ref):
    acc = (ap_ref[0] + ap_ref[1]
           + jnp.dot(x_ref[...], root_ref[...],
                     preferred_element_type=jnp.float32)
           + b_ref[...])
    o_ref[...] = jnp.maximum(acc, 0.0)


def _comb_call(aggp, x, root, bias):
    return pl.pallas_call(
        _comb_body,
        grid=(N // BN,),
        in_specs=[pl.BlockSpec((2, BN, D), lambda i: (0, i, 0)),
                  pl.BlockSpec((BN, D), lambda i: (i, 0)),
                  pl.BlockSpec((D, D), lambda i: (0, 0)),
                  pl.BlockSpec((1, D), lambda i: (0, 0))],
        out_specs=pl.BlockSpec((BN, D), lambda i: (i, 0)),
        out_shape=jax.ShapeDtypeStruct((N, D), jnp.float32),
    )(aggp, x, root, bias.reshape(1, D))


def kernel(ent_emb, comp0, bases0, root0, bias0, comp1, bases1, root1, bias1,
           edge_index, edge_type):
    src = edge_index[0].astype(jnp.int32)
    dst = edge_index[1].astype(jnp.int32)
    et = edge_type.astype(jnp.int32)
    pad = E_PAD - E
    trash_pair = NR + (jnp.arange(pad, dtype=jnp.int32) % (NR_PAD - NR))
    trash_dst = N + (jnp.arange(pad, dtype=jnp.int32) % (N_PAD - N))
    gidx = jnp.concatenate([src * R + et,
                            jnp.zeros((pad,), jnp.int32)]).reshape(
                                NCHUNK, NSUB, 128)
    npair = jnp.concatenate([dst * R + et,
                             trash_pair]).reshape(NCHUNK, NSUB, 128)
    dsti = jnp.concatenate([dst, trash_dst]).reshape(NCHUNK, NSUB, 128)

    idxall = jnp.concatenate([gidx, npair, dsti], axis=1)

    cntp = _cnt_call(npair)
    norm = 1.0 / jnp.maximum(cntp[0] + cntp[1], 1.0)

    x = ent_emb
    for comp, bases, root, bias in ((comp0, bases0, root0, bias0),
                                    (comp1, bases1, root1, bias1)):
        wcat = jnp.einsum('rb,bio->iro', comp, bases).reshape(D, R * D)
        h = _h_call(x, wcat[:, _HPERM]).reshape(NR, D)
        aggp = _agg_call(idxall, h, norm)
        x = _comb_call(aggp, x, root, bias)
    return x
